# Initial kernel scaffold; baseline (speedup 1.0000x reference)
import jax
import jax.numpy as jnp
from jax.experimental import pallas as pl


def kernel(atom_types, edge_index, batch_ids, distances, emb, node_W, node_b, edge_W, edge_b, msg_W1, msg_b1, msg_W2, msg_b2, upd_W1, upd_b1, upd_W2, upd_b2, ln_g, ln_b, pW1, pb1, pW2, pb2, pW3, pb3, pW4, pb4):
    raise NotImplementedError("write your pallas kernel here")


# speedup vs baseline: 1.3822x; 1.3822x over previous
import functools

import jax
import jax.numpy as jnp
from jax import lax
from jax.experimental import pallas as pl
from jax.experimental.pallas import tpu as pltpu
from jax.experimental.pallas import tpu_sc as plsc

F32 = jnp.float32
I32 = jnp.int32

_NS = 16
_W = 128


def _edge_pass_kernel(N, E, D):
    EPT = E // _NS
    NWIN = EPT // _W
    TAIL = EPT - NWIN * _W
    PSTR = (N // _NS) & ~7
    PEXT = N - _NS * PSTR
    NQ = D // 16
    mesh = plsc.VectorSubcoreMesh(core_axis_name="c", subcore_axis_name="s")

    out_type = [jax.ShapeDtypeStruct((N, D), F32) for _ in range(2)]

    scratch = [
        pltpu.VMEM_SHARED((N, D), F32),
        pltpu.VMEM((_W,), I32),
        pltpu.VMEM((_W,), I32),
        pltpu.VMEM((_W,), F32),
        pltpu.VMEM((_W, D), F32),
        pltpu.VMEM((_W, D), F32),
        pltpu.VMEM((16, D), F32),
        pltpu.SemaphoreType.DMA,
        pltpu.SemaphoreType.DMA,
    ]
    if TAIL:
        scratch += [
            pltpu.VMEM((TAIL,), I32),
            pltpu.VMEM((TAIL,), I32),
            pltpu.VMEM((TAIL, D), F32),
            pltpu.VMEM((TAIL, D), F32),
        ]

    @functools.partial(pl.kernel, mesh=mesh, out_type=out_type,
                       scratch_types=scratch,
                       compiler_params=pltpu.CompilerParams(
                           needs_layout_passes=False))
    def kern(dst_h, src_h, dist_h, a_h, b_h, *rest):
        it = iter(rest)
        o0, o1 = next(it), next(it)
        acc = next(it)
        idxd, idxs, dvec, rowsA, rowsB, zbuf = (
            next(it), next(it), next(it), next(it), next(it), next(it))
        semA, semB = next(it), next(it)
        if TAIL:
            idxd_t, idxs_t, rowsA_t, rowsB_t = (
                next(it), next(it), next(it), next(it))

        c = lax.axis_index("c")
        s = lax.axis_index("s")
        zero16 = jnp.zeros((16,), F32)
        c_is1 = jnp.full((16,), c, I32) == 1

        def zrow(r, _):
            for q in range(NQ):
                zbuf[r, pl.ds(16 * q, 16)] = zero16
            return 0
        lax.fori_loop(0, 16, zrow, 0)
        stripe = pl.ds(s * PSTR, PSTR)
        ext = pl.ds(_NS * PSTR, PEXT)

        def zcp(k, _):
            pltpu.sync_copy(zbuf, acc.at[pl.ds(s * PSTR + 16 * k, 16)])
            return 0
        lax.fori_loop(0, PSTR // 16, zcp, 0)
        if PEXT:
            @pl.when(s == _NS - 1)
            def _():
                pltpu.sync_copy(zbuf.at[pl.ds(0, PEXT)], acc.at[ext])
        plsc.subcore_barrier()

        def window(start, w, _idxd, _idxs, _rowsA, _rowsB):
            pltpu.sync_copy(dst_h.at[pl.ds(start, w)], _idxd)
            pltpu.sync_copy(src_h.at[pl.ds(start, w)], _idxs)
            pltpu.sync_copy(dist_h.at[pl.ds(start, w)], dvec.at[pl.ds(0, w)])
            cpA = pltpu.async_copy(a_h.at[_idxd], _rowsA, semA)
            cpB = pltpu.async_copy(b_h.at[_idxs], _rowsB, semB)
            cpA.wait()
            cpB.wait()

            def row(r, _):
                dspl = plsc.load_gather(dvec, [jnp.zeros((16,), I32) + r])
                scale = jnp.where(c_is1, dspl, 1.0)
                for q in range(NQ):
                    sl = pl.ds(16 * q, 16)
                    h = jnp.maximum(_rowsA[r, sl] + _rowsB[r, sl], 0.0)
                    _rowsA[r, sl] = h * scale
                return 0
            lax.fori_loop(0, w, row, 0)

            pltpu.sync_copy(_rowsA, acc.at[_idxd], add=True)

        base = s * EPT

        def win_body(k, _):
            window(base + k * _W, _W, idxd, idxs, rowsA, rowsB)
            return 0
        lax.fori_loop(0, NWIN, win_body, 0)
        if TAIL:
            window(base + NWIN * _W, TAIL, idxd_t, idxs_t, rowsA_t, rowsB_t)

        plsc.subcore_barrier()

        @pl.when(c == 0)
        def _():
            pltpu.sync_copy(acc.at[stripe], o0.at[stripe])
            if PEXT:
                @pl.when(s == _NS - 1)
                def _():
                    pltpu.sync_copy(acc.at[ext], o0.at[ext])

        @pl.when(c == 1)
        def _():
            pltpu.sync_copy(acc.at[stripe], o1.at[stripe])
            if PEXT:
                @pl.when(s == _NS - 1)
                def _():
                    pltpu.sync_copy(acc.at[ext], o1.at[ext])

    return kern


def _prep0_call(at_f32, emb, nW, nb, W1a, W1b, b1, N, D, V):
    def body(at_ref, emb_ref, nW_ref, nb_ref, W1a_ref, W1b_ref, b1_ref,
             x_ref, a_ref, b_ref):
        at = at_ref[...]
        vid = lax.broadcasted_iota(I32, (1, V), 1).astype(F32)
        onehot = jnp.where(at == vid + 1.0, 1.0, 0.0)
        x = jnp.dot(onehot, emb_ref[...], preferred_element_type=F32)
        xt = jnp.dot(x, nW_ref[...], preferred_element_type=F32) + nb_ref[...]
        A = jnp.dot(xt, W1a_ref[...], preferred_element_type=F32) + b1_ref[...]
        B = jnp.dot(xt, W1b_ref[...], preferred_element_type=F32)
        x_ref[...] = x
        a_ref[...] = A
        b_ref[...] = B

    outs = [jax.ShapeDtypeStruct((N, D), F32)] * 3
    return pl.pallas_call(body, out_shape=outs)(
        at_f32, emb, nW, nb, W1a, W1b, b1)


def _update_call(x, s0, s1, deg, sd, lw, residual, nxt, N, D):
    last = nxt is None

    def body(*refs):
        it = iter(refs)
        x_ref, s0r, s1r, degr, sdr = (next(it) for _ in range(5))
        W2r, wr, ber, c2r, U1ar, U1br, ub1r, U2r, ub2r, gr, br = (
            next(it) for _ in range(11))
        if last:
            bidr, pW1r, pb1r, pW2r, pb2r, pW3r, pb3r, pW4r, pb4r = (
                next(it) for _ in range(9))
            out_ref = next(it)
        else:
            nWr, nbr, W1ar, W1br, b1r = (next(it) for _ in range(5))
            xo, ao, bo = (next(it) for _ in range(3))

        x_ = x_ref[...]
        s0W = jnp.dot(s0r[...], W2r[...], preferred_element_type=F32)
        s1W = jnp.dot(s1r[...], W2r[...], preferred_element_type=F32)
        w_ = wr[...]; be = ber[...]; c2 = c2r[...]
        agg = (s1W * w_ + s0W * be +
               sdr[...] * (c2 * w_) + degr[...] * (c2 * be))
        u = jnp.maximum(jnp.dot(agg, U1ar[...], preferred_element_type=F32) +
                        jnp.dot(x_, U1br[...], preferred_element_type=F32) +
                        ub1r[...], 0.0)
        u = jnp.maximum(jnp.dot(u, U2r[...], preferred_element_type=F32) +
                        ub2r[...], 0.0)
        mu = jnp.mean(u, axis=-1, keepdims=True)
        d = u - mu
        var = jnp.mean(d * d, axis=-1, keepdims=True)
        u = d * jax.lax.rsqrt(var + 1e-5) * gr[...] + br[...]
        xn = u if not residual else x_ + u

        if last:
            bid = bidr[...]
            gid = lax.broadcasted_iota(I32, (1, 16), 1).astype(F32)
            ohg = jnp.where(bid == gid, 1.0, 0.0)
            sums = lax.dot_general(ohg, xn, (((0,), (0,)), ((), ())),
                                   preferred_element_type=F32)
            ones = jnp.zeros((N, 1), F32) + 1.0
            counts = lax.dot_general(ohg, ones, (((0,), (0,)), ((), ())),
                                     preferred_element_type=F32)
            pooled = sums / jnp.maximum(counts, 1.0)
            h = jnp.maximum(jnp.dot(pooled, pW1r[...],
                                    preferred_element_type=F32) + pb1r[...], 0.0)
            h = jnp.maximum(jnp.dot(h, pW2r[...],
                                    preferred_element_type=F32) + pb2r[...], 0.0)
            h = jnp.maximum(jnp.dot(h, pW3r[...],
                                    preferred_element_type=F32) + pb3r[...], 0.0)
            out_ref[...] = jnp.dot(h, pW4r[...],
                                   preferred_element_type=F32) + pb4r[...]
        else:
            xt = jnp.dot(xn, nWr[...], preferred_element_type=F32) + nbr[...]
            A = jnp.dot(xt, W1ar[...], preferred_element_type=F32) + b1r[...]
            B = jnp.dot(xt, W1br[...], preferred_element_type=F32)
            xo[...] = xn
            ao[...] = A
            bo[...] = B

    args = [x, s0, s1, deg, sd,
            lw["W2"], lw["w"], lw["be"], lw["c2"], lw["U1a"], lw["U1b"],
            lw["ub1"], lw["U2"], lw["ub2"], lw["g"], lw["b"]]
    if last:
        args += [lw["bid"], lw["pW1"], lw["pb1"], lw["pW2"], lw["pb2"],
                 lw["pW3"], lw["pb3"], lw["pW4"], lw["pb4"]]
        outs = jax.ShapeDtypeStruct((16, 1), F32)
    else:
        args += [nxt["nW"], nxt["nb"], nxt["W1a"], nxt["W1b"], nxt["b1"]]
        outs = [jax.ShapeDtypeStruct((N, D), F32)] * 3
    return pl.pallas_call(body, out_shape=outs)(*args)


def kernel(atom_types, edge_index, batch_ids, distances, emb, node_W, node_b,
           edge_W, edge_b, msg_W1, msg_b1, msg_W2, msg_b2, upd_W1, upd_b1,
           upd_W2, upd_b2, ln_g, ln_b, pW1, pb1, pW2, pb2, pW3, pb3, pW4, pb4):
    N = atom_types.shape[0]
    E = distances.shape[0]
    V, D = emb.shape
    L = node_W.shape[0]

    at_f32 = atom_types.astype(F32).reshape(N, 1)
    bid_f32 = batch_ids.astype(F32).reshape(N, 1)
    src = edge_index[0]
    dst = edge_index[1]
    dist = distances.astype(F32)

    deg = jax.ops.segment_sum(jnp.ones((E,), F32), dst,
                              num_segments=N).reshape(N, 1)
    sd = jax.ops.segment_sum(dist, dst, num_segments=N).reshape(N, 1)

    row = lambda v: v.reshape(1, -1)

    def layer_w(i):
        return dict(
            W2=msg_W2[i], w=row(edge_W[i][0]), be=row(edge_b[i]),
            c2=row(msg_b2[i]), U1a=upd_W1[i][:D], U1b=upd_W1[i][D:],
            ub1=row(upd_b1[i]), U2=upd_W2[i], ub2=row(upd_b2[i]),
            g=row(ln_g[i]), b=row(ln_b[i]))

    def prep_w(i):
        return dict(nW=node_W[i], nb=row(node_b[i]), W1a=msg_W1[i][:D],
                    W1b=msg_W1[i][D:], b1=row(msg_b1[i]))

    p0 = prep_w(0)
    x, a_t, b_t = _prep0_call(at_f32, emb, p0["nW"], p0["nb"],
                              p0["W1a"], p0["W1b"], p0["b1"], N, D, V)

    edge = _edge_pass_kernel(N, E, D)

    for i in range(L):
        s0, s1 = edge(dst, src, dist, a_t, b_t)
        lw = layer_w(i)
        if i < L - 1:
            x, a_t, b_t = _update_call(x, s0, s1, deg, sd, lw,
                                       residual=(i > 0), nxt=prep_w(i + 1),
                                       N=N, D=D)
        else:
            lw.update(bid=bid_f32, pW1=pW1, pb1=row(pb1), pW2=pW2,
                      pb2=row(pb2), pW3=pW3, pb3=row(pb3), pW4=pW4,
                      pb4=row(pb4))
            out = _update_call(x, s0, s1, deg, sd, lw, residual=True,
                               nxt=None, N=N, D=D)
    return out


# speedup vs baseline: 1.4643x; 1.0594x over previous
import functools

import jax
import jax.numpy as jnp
from jax import lax
from jax.experimental import pallas as pl
from jax.experimental.pallas import tpu as pltpu
from jax.experimental.pallas import tpu_sc as plsc

F32 = jnp.float32
I32 = jnp.int32

_NS = 16
_W = 64


def _edge_pass_kernel(N, E, D):
    EPT = E // _NS
    NWIN = EPT // _W
    TAIL = EPT - NWIN * _W
    PSTR = (N // _NS) & ~7
    PEXT = N - _NS * PSTR
    NQ = D // 16
    assert NWIN % 2 == 0 and NWIN >= 4
    mesh = plsc.VectorSubcoreMesh(core_axis_name="c", subcore_axis_name="s")

    out_type = [jax.ShapeDtypeStruct((N, D), F32) for _ in range(2)]

    scratch = [
        pltpu.VMEM_SHARED((N, D), F32),
        pltpu.VMEM((_W,), I32),
        pltpu.VMEM((_W,), I32),
        pltpu.VMEM((_W,), F32),
        pltpu.VMEM((_W,), I32),
        pltpu.VMEM((_W,), I32),
        pltpu.VMEM((_W,), F32),
        pltpu.VMEM((_W, D), F32),
        pltpu.VMEM((_W, D), F32),
        pltpu.VMEM((_W, D), F32),
        pltpu.VMEM((_W, D), F32),
        pltpu.VMEM((16, D), F32),
        pltpu.SemaphoreType.DMA,
        pltpu.SemaphoreType.DMA,
        pltpu.SemaphoreType.DMA,
        pltpu.SemaphoreType.DMA,
        pltpu.SemaphoreType.DMA,
        pltpu.SemaphoreType.DMA,
    ]
    if TAIL:
        scratch += [
            pltpu.VMEM((TAIL,), I32),
            pltpu.VMEM((TAIL,), I32),
            pltpu.VMEM((TAIL,), F32),
            pltpu.VMEM((TAIL, D), F32),
            pltpu.VMEM((TAIL, D), F32),
        ]

    @functools.partial(pl.kernel, mesh=mesh, out_type=out_type,
                       scratch_types=scratch,
                       compiler_params=pltpu.CompilerParams(
                           needs_layout_passes=False))
    def kern(dst_h, src_h, dist_h, a_h, b_h, *rest):
        it = iter(rest)
        o0, o1 = next(it), next(it)
        acc = next(it)
        idxd0, idxs0, dvec0, idxd1, idxs1, dvec1 = (
            next(it), next(it), next(it), next(it), next(it), next(it))
        rowsA0, rowsB0, rowsA1, rowsB1, zbuf = (
            next(it), next(it), next(it), next(it), next(it))
        semA0, semB0, semA1, semB1, semS0, semS1 = (
            next(it), next(it), next(it), next(it), next(it), next(it))
        if TAIL:
            idxd_t, idxs_t, dvec_t, rowsA_t, rowsB_t = (
                next(it), next(it), next(it), next(it), next(it))

        c = lax.axis_index("c")
        s = lax.axis_index("s")
        zero16 = jnp.zeros((16,), F32)
        c_is1 = jnp.full((16,), c, I32) == 1

        BUF = ((idxd0, idxs0, dvec0, rowsA0, rowsB0, semA0, semB0, semS0),
               (idxd1, idxs1, dvec1, rowsA1, rowsB1, semA1, semB1, semS1))

        def zrow(r, _):
            for q in range(NQ):
                zbuf[r, pl.ds(16 * q, 16)] = zero16
            return 0
        lax.fori_loop(0, 16, zrow, 0)
        stripe = pl.ds(s * PSTR, PSTR)
        ext = pl.ds(_NS * PSTR, PEXT)

        def zcp(k, _):
            pltpu.sync_copy(zbuf, acc.at[pl.ds(s * PSTR + 16 * k, 16)])
            return 0
        lax.fori_loop(0, PSTR // 16, zcp, 0)
        if PEXT:
            @pl.when(s == _NS - 1)
            def _():
                pltpu.sync_copy(zbuf.at[pl.ds(0, PEXT)], acc.at[ext])
        plsc.subcore_barrier()

        base = s * EPT

        def fire(k, b):
            _id, _is, _dv, _ra, _rb, _sa, _sb, _ss = BUF[b]
            start = base + k * _W
            pltpu.sync_copy(dst_h.at[pl.ds(start, _W)], _id)
            pltpu.sync_copy(src_h.at[pl.ds(start, _W)], _is)
            pltpu.sync_copy(dist_h.at[pl.ds(start, _W)], _dv)
            pltpu.async_copy(a_h.at[_id], _ra, _sa)
            pltpu.async_copy(b_h.at[_is], _rb, _sb)

        def fire_guarded(k, b):
            @pl.when(k < NWIN)
            def _():
                pltpu.make_async_copy(BUF[b][3], acc.at[BUF[b][0]],
                                      BUF[b][7]).wait()
                fire(k, b)

        def compute(w, _dv, _ra, _rb):
            def row(r, _):
                dspl = plsc.load_gather(_dv, [jnp.zeros((16,), I32) + r])
                scale = jnp.where(c_is1, dspl, 1.0)
                for q in range(NQ):
                    sl = pl.ds(16 * q, 16)
                    h = jnp.maximum(_ra[r, sl] + _rb[r, sl], 0.0)
                    _ra[r, sl] = h * scale
                return 0
            lax.fori_loop(0, w, row, 0)

        def finish(b):
            _id, _is, _dv, _ra, _rb, _sa, _sb, _ss = BUF[b]
            pltpu.make_async_copy(a_h.at[_id], _ra, _sa).wait()
            pltpu.make_async_copy(b_h.at[_is], _rb, _sb).wait()
            compute(_W, _dv, _ra, _rb)
            pltpu.async_copy(_ra, acc.at[_id], _ss, add=True)

        fire(0, 0)
        fire(1, 1)

        def pair(p, _):
            finish(0)
            fire_guarded(2 * p + 2, 0)
            finish(1)
            fire_guarded(2 * p + 3, 1)
            return 0
        lax.fori_loop(0, NWIN // 2, pair, 0)
        pltpu.make_async_copy(BUF[0][3], acc.at[BUF[0][0]], BUF[0][7]).wait()
        pltpu.make_async_copy(BUF[1][3], acc.at[BUF[1][0]], BUF[1][7]).wait()

        if TAIL:
            start = base + NWIN * _W
            pltpu.sync_copy(dst_h.at[pl.ds(start, TAIL)], idxd_t)
            pltpu.sync_copy(src_h.at[pl.ds(start, TAIL)], idxs_t)
            pltpu.sync_copy(dist_h.at[pl.ds(start, TAIL)], dvec_t)
            cpA = pltpu.async_copy(a_h.at[idxd_t], rowsA_t, semA0)
            cpB = pltpu.async_copy(b_h.at[idxs_t], rowsB_t, semB0)
            cpA.wait()
            cpB.wait()
            compute(TAIL, dvec_t, rowsA_t, rowsB_t)
            pltpu.sync_copy(rowsA_t, acc.at[idxd_t], add=True)

        plsc.subcore_barrier()

        @pl.when(c == 0)
        def _():
            pltpu.sync_copy(acc.at[stripe], o0.at[stripe])
            if PEXT:
                @pl.when(s == _NS - 1)
                def _():
                    pltpu.sync_copy(acc.at[ext], o0.at[ext])

        @pl.when(c == 1)
        def _():
            pltpu.sync_copy(acc.at[stripe], o1.at[stripe])
            if PEXT:
                @pl.when(s == _NS - 1)
                def _():
                    pltpu.sync_copy(acc.at[ext], o1.at[ext])

    return kern


def _prep0_call(at_f32, emb, nW, nb, W1a, W1b, b1, N, D, V):
    def body(at_ref, emb_ref, nW_ref, nb_ref, W1a_ref, W1b_ref, b1_ref,
             x_ref, a_ref, b_ref):
        at = at_ref[...]
        vid = lax.broadcasted_iota(I32, (1, V), 1).astype(F32)
        onehot = jnp.where(at == vid + 1.0, 1.0, 0.0)
        x = jnp.dot(onehot, emb_ref[...], preferred_element_type=F32)
        xt = jnp.dot(x, nW_ref[...], preferred_element_type=F32) + nb_ref[...]
        A = jnp.dot(xt, W1a_ref[...], preferred_element_type=F32) + b1_ref[...]
        B = jnp.dot(xt, W1b_ref[...], preferred_element_type=F32)
        x_ref[...] = x
        a_ref[...] = A
        b_ref[...] = B

    outs = [jax.ShapeDtypeStruct((N, D), F32)] * 3
    return pl.pallas_call(body, out_shape=outs)(
        at_f32, emb, nW, nb, W1a, W1b, b1)


def _update_call(x, s0, s1, deg, sd, lw, residual, nxt, N, D):
    last = nxt is None

    def body(*refs):
        it = iter(refs)
        x_ref, s0r, s1r, degr, sdr = (next(it) for _ in range(5))
        W2r, wr, ber, c2r, U1ar, U1br, ub1r, U2r, ub2r, gr, br = (
            next(it) for _ in range(11))
        if last:
            bidr, pW1r, pb1r, pW2r, pb2r, pW3r, pb3r, pW4r, pb4r = (
                next(it) for _ in range(9))
            out_ref = next(it)
        else:
            nWr, nbr, W1ar, W1br, b1r = (next(it) for _ in range(5))
            xo, ao, bo = (next(it) for _ in range(3))

        x_ = x_ref[...]
        s0W = jnp.dot(s0r[...], W2r[...], preferred_element_type=F32)
        s1W = jnp.dot(s1r[...], W2r[...], preferred_element_type=F32)
        w_ = wr[...]; be = ber[...]; c2 = c2r[...]
        agg = (s1W * w_ + s0W * be +
               sdr[...] * (c2 * w_) + degr[...] * (c2 * be))
        u = jnp.maximum(jnp.dot(agg, U1ar[...], preferred_element_type=F32) +
                        jnp.dot(x_, U1br[...], preferred_element_type=F32) +
                        ub1r[...], 0.0)
        u = jnp.maximum(jnp.dot(u, U2r[...], preferred_element_type=F32) +
                        ub2r[...], 0.0)
        mu = jnp.mean(u, axis=-1, keepdims=True)
        d = u - mu
        var = jnp.mean(d * d, axis=-1, keepdims=True)
        u = d * jax.lax.rsqrt(var + 1e-5) * gr[...] + br[...]
        xn = u if not residual else x_ + u

        if last:
            bid = bidr[...]
            gid = lax.broadcasted_iota(I32, (1, 16), 1).astype(F32)
            ohg = jnp.where(bid == gid, 1.0, 0.0)
            sums = lax.dot_general(ohg, xn, (((0,), (0,)), ((), ())),
                                   preferred_element_type=F32)
            ones = jnp.zeros((N, 1), F32) + 1.0
            counts = lax.dot_general(ohg, ones, (((0,), (0,)), ((), ())),
                                     preferred_element_type=F32)
            pooled = sums / jnp.maximum(counts, 1.0)
            h = jnp.maximum(jnp.dot(pooled, pW1r[...],
                                    preferred_element_type=F32) + pb1r[...], 0.0)
            h = jnp.maximum(jnp.dot(h, pW2r[...],
                                    preferred_element_type=F32) + pb2r[...], 0.0)
            h = jnp.maximum(jnp.dot(h, pW3r[...],
                                    preferred_element_type=F32) + pb3r[...], 0.0)
            out_ref[...] = jnp.dot(h, pW4r[...],
                                   preferred_element_type=F32) + pb4r[...]
        else:
            xt = jnp.dot(xn, nWr[...], preferred_element_type=F32) + nbr[...]
            A = jnp.dot(xt, W1ar[...], preferred_element_type=F32) + b1r[...]
            B = jnp.dot(xt, W1br[...], preferred_element_type=F32)
            xo[...] = xn
            ao[...] = A
            bo[...] = B

    args = [x, s0, s1, deg, sd,
            lw["W2"], lw["w"], lw["be"], lw["c2"], lw["U1a"], lw["U1b"],
            lw["ub1"], lw["U2"], lw["ub2"], lw["g"], lw["b"]]
    if last:
        args += [lw["bid"], lw["pW1"], lw["pb1"], lw["pW2"], lw["pb2"],
                 lw["pW3"], lw["pb3"], lw["pW4"], lw["pb4"]]
        outs = jax.ShapeDtypeStruct((16, 1), F32)
    else:
        args += [nxt["nW"], nxt["nb"], nxt["W1a"], nxt["W1b"], nxt["b1"]]
        outs = [jax.ShapeDtypeStruct((N, D), F32)] * 3
    return pl.pallas_call(body, out_shape=outs)(*args)


def kernel(atom_types, edge_index, batch_ids, distances, emb, node_W, node_b,
           edge_W, edge_b, msg_W1, msg_b1, msg_W2, msg_b2, upd_W1, upd_b1,
           upd_W2, upd_b2, ln_g, ln_b, pW1, pb1, pW2, pb2, pW3, pb3, pW4, pb4):
    N = atom_types.shape[0]
    E = distances.shape[0]
    V, D = emb.shape
    L = node_W.shape[0]

    at_f32 = atom_types.astype(F32).reshape(N, 1)
    bid_f32 = batch_ids.astype(F32).reshape(N, 1)
    src = edge_index[0]
    dst = edge_index[1]
    dist = distances.astype(F32)

    deg = jax.ops.segment_sum(jnp.ones((E,), F32), dst,
                              num_segments=N).reshape(N, 1)
    sd = jax.ops.segment_sum(dist, dst, num_segments=N).reshape(N, 1)

    row = lambda v: v.reshape(1, -1)

    def layer_w(i):
        return dict(
            W2=msg_W2[i], w=row(edge_W[i][0]), be=row(edge_b[i]),
            c2=row(msg_b2[i]), U1a=upd_W1[i][:D], U1b=upd_W1[i][D:],
            ub1=row(upd_b1[i]), U2=upd_W2[i], ub2=row(upd_b2[i]),
            g=row(ln_g[i]), b=row(ln_b[i]))

    def prep_w(i):
        return dict(nW=node_W[i], nb=row(node_b[i]), W1a=msg_W1[i][:D],
                    W1b=msg_W1[i][D:], b1=row(msg_b1[i]))

    p0 = prep_w(0)
    x, a_t, b_t = _prep0_call(at_f32, emb, p0["nW"], p0["nb"],
                              p0["W1a"], p0["W1b"], p0["b1"], N, D, V)

    edge = _edge_pass_kernel(N, E, D)

    for i in range(L):
        s0, s1 = edge(dst, src, dist, a_t, b_t)
        lw = layer_w(i)
        if i < L - 1:
            x, a_t, b_t = _update_call(x, s0, s1, deg, sd, lw,
                                       residual=(i > 0), nxt=prep_w(i + 1),
                                       N=N, D=D)
        else:
            lw.update(bid=bid_f32, pW1=pW1, pb1=row(pb1), pW2=pW2,
                      pb2=row(pb2), pW3=pW3, pb3=row(pb3), pW4=pW4,
                      pb4=row(pb4))
            out = _update_call(x, s0, s1, deg, sd, lw, residual=True,
                               nxt=None, N=N, D=D)
    return out


# speedup vs baseline: 1.5727x; 1.0740x over previous
import functools

import jax
import jax.numpy as jnp
from jax import lax
from jax.experimental import pallas as pl
from jax.experimental.pallas import tpu as pltpu
from jax.experimental.pallas import tpu_sc as plsc

F32 = jnp.float32
I32 = jnp.int32

_NS = 16
_W = 64
_CPW = 8


def _edge_pass_kernel(N, NP, E2, D):
    NWIN = E2 // (_NS * _W)
    NCH = NWIN // _CPW
    PSTR = (N // _NS) & ~7
    PEXT = N - _NS * PSTR
    NQ = D // 16
    assert NWIN % _CPW == 0 and NCH % 2 == 0 and NCH >= 4
    mesh = plsc.VectorSubcoreMesh(core_axis_name="c", subcore_axis_name="s")

    out_type = [jax.ShapeDtypeStruct((N, D), F32) for _ in range(2)]

    scratch = [
        pltpu.VMEM_SHARED((NP, D), F32),
        pltpu.VMEM((_CPW, _W), I32),
        pltpu.VMEM((_CPW, _W), I32),
        pltpu.VMEM((_CPW, _W), F32),
        pltpu.VMEM((_CPW, _W), I32),
        pltpu.VMEM((_CPW, _W), I32),
        pltpu.VMEM((_CPW, _W), F32),
        pltpu.VMEM((_W, D), F32),
        pltpu.VMEM((_W, D), F32),
        pltpu.VMEM((_W, D), F32),
        pltpu.VMEM((_W, D), F32),
        pltpu.VMEM((16, D), F32),
        pltpu.SemaphoreType.DMA,
        pltpu.SemaphoreType.DMA,
        pltpu.SemaphoreType.DMA,
        pltpu.SemaphoreType.DMA,
        pltpu.SemaphoreType.DMA,
        pltpu.SemaphoreType.DMA,
        pltpu.SemaphoreType.DMA,
        pltpu.SemaphoreType.DMA,
    ]

    @functools.partial(pl.kernel, mesh=mesh, out_type=out_type,
                       scratch_types=scratch,
                       compiler_params=pltpu.CompilerParams(
                           needs_layout_passes=False))
    def kern(dst_h, src_h, dist_h, a_h, b_h, *rest):
        it = iter(rest)
        o0, o1 = next(it), next(it)
        acc = next(it)
        icd0, ics0, icv0, icd1, ics1, icv1 = (
            next(it), next(it), next(it), next(it), next(it), next(it))
        rowsA0, rowsB0, rowsA1, rowsB1, zbuf = (
            next(it), next(it), next(it), next(it), next(it))
        semA0, semB0, semA1, semB1, semS0, semS1, semI0, semI1 = (
            next(it), next(it), next(it), next(it), next(it), next(it),
            next(it), next(it))

        c = lax.axis_index("c")
        s = lax.axis_index("s")
        zero16 = jnp.zeros((16,), F32)
        c_is1 = jnp.full((16,), c, I32) == 1

        ICD, ICS, ICV = (icd0, icd1), (ics0, ics1), (icv0, icv1)
        SEMI = (semI0, semI1)
        ROWSA, ROWSB = (rowsA0, rowsA1), (rowsB0, rowsB1)
        SEMA, SEMB, SEMS = (semA0, semA1), (semB0, semB1), (semS0, semS1)

        def zrow(r, _):
            for q in range(NQ):
                zbuf[r, pl.ds(16 * q, 16)] = zero16
            return 0
        lax.fori_loop(0, 16, zrow, 0)
        stripe = pl.ds(s * PSTR, PSTR)
        ext = pl.ds(_NS * PSTR, PEXT)

        def zcp(k, _):
            pltpu.sync_copy(zbuf, acc.at[pl.ds(s * PSTR + 16 * k, 16)])
            return 0
        lax.fori_loop(0, PSTR // 16, zcp, 0)
        if PEXT:
            @pl.when(s == _NS - 1)
            def _():
                pltpu.sync_copy(zbuf.at[pl.ds(0, PEXT)], acc.at[ext])
        plsc.subcore_barrier()

        rbase = s * NWIN

        def fire_chunk(cc, ib):
            rows = pl.ds(rbase + cc * _CPW, _CPW)
            pltpu.async_copy(dst_h.at[rows], ICD[ib], SEMI[ib])
            pltpu.async_copy(src_h.at[rows], ICS[ib], SEMI[ib])
            pltpu.async_copy(dist_h.at[rows], ICV[ib], SEMI[ib])

        def drain_chunk(ib):
            rows = pl.ds(0, _CPW)
            pltpu.make_async_copy(dst_h.at[rows], ICD[ib], SEMI[ib]).wait()
            pltpu.make_async_copy(src_h.at[rows], ICS[ib], SEMI[ib]).wait()
            pltpu.make_async_copy(dist_h.at[rows], ICV[ib], SEMI[ib]).wait()

        def compute(_dv, _ra, _rb):
            def row(r, _):
                dspl = plsc.load_gather(_dv, [jnp.zeros((16,), I32) + r])
                scale = jnp.where(c_is1, dspl, 1.0)
                for q in range(NQ):
                    sl = pl.ds(16 * q, 16)
                    h = jnp.maximum(_ra[r, sl] + _rb[r, sl], 0.0)
                    _ra[r, sl] = h * scale
                return 0
            lax.fori_loop(0, _W, row, 0)

        def win_fire(ib, j, first=False):
            rb = j % 2
            if not first:
                pltpu.make_async_copy(ROWSA[rb], acc.at[ICD[0].at[0]],
                                      SEMS[rb]).wait()
            pltpu.async_copy(a_h.at[ICD[ib].at[j]], ROWSA[rb], SEMA[rb])
            pltpu.async_copy(b_h.at[ICS[ib].at[j]], ROWSB[rb], SEMB[rb])

        def win_complete(ib, j):
            rb = j % 2
            pltpu.make_async_copy(a_h.at[ICD[ib].at[j]], ROWSA[rb],
                                  SEMA[rb]).wait()
            pltpu.make_async_copy(b_h.at[ICS[ib].at[j]], ROWSB[rb],
                                  SEMB[rb]).wait()
            compute(ICV[ib].at[j], ROWSA[rb], ROWSB[rb])
            pltpu.async_copy(ROWSA[rb], acc.at[ICD[ib].at[j]], SEMS[rb],
                             add=True)

        def chunk_body(cc, ib, peel0):
            drain_chunk(ib)
            for j in range(_CPW):
                if peel0 and j < 2:
                    pass
                elif j < 2:
                    win_complete(ib ^ 1, _CPW - 2 + j)
                else:
                    win_complete(ib, j - 2)
                if j == 2:
                    @pl.when(cc + 1 < NCH)
                    def _():
                        fire_chunk(cc + 1, ib ^ 1)
                win_fire(ib, j, first=(peel0 and j < 2))

        fire_chunk(0, 0)
        chunk_body(0, 0, True)
        chunk_body(1, 1, False)

        def pairbody(p, _):
            chunk_body(2 * p + 2, 0, False)
            chunk_body(2 * p + 3, 1, False)
            return 0
        lax.fori_loop(0, (NCH - 2) // 2, pairbody, 0)

        win_complete(1, _CPW - 2)
        win_complete(1, _CPW - 1)
        pltpu.make_async_copy(ROWSA[0], acc.at[ICD[0].at[0]], SEMS[0]).wait()
        pltpu.make_async_copy(ROWSA[1], acc.at[ICD[0].at[0]], SEMS[1]).wait()

        plsc.subcore_barrier()

        @pl.when(c == 0)
        def _():
            pltpu.sync_copy(acc.at[stripe], o0.at[stripe])
            if PEXT:
                @pl.when(s == _NS - 1)
                def _():
                    pltpu.sync_copy(acc.at[ext], o0.at[ext])

        @pl.when(c == 1)
        def _():
            pltpu.sync_copy(acc.at[stripe], o1.at[stripe])
            if PEXT:
                @pl.when(s == _NS - 1)
                def _():
                    pltpu.sync_copy(acc.at[ext], o1.at[ext])

    return kern


def _prep0_call(at_f32, emb, nW, nb, W1a, W1b, b1, N, NP, D, V):
    def body(at_ref, emb_ref, nW_ref, nb_ref, W1a_ref, W1b_ref, b1_ref,
             x_ref, a_ref, b_ref):
        at = at_ref[...]
        vid = lax.broadcasted_iota(I32, (1, V), 1).astype(F32)
        onehot = jnp.where(at == vid + 1.0, 1.0, 0.0)
        x = jnp.dot(onehot, emb_ref[...], preferred_element_type=F32)
        xt = jnp.dot(x, nW_ref[...], preferred_element_type=F32) + nb_ref[...]
        A = jnp.dot(xt, W1a_ref[...], preferred_element_type=F32) + b1_ref[...]
        B = jnp.dot(xt, W1b_ref[...], preferred_element_type=F32)
        pad = jnp.zeros((NP - N, D), F32) - 1e9
        x_ref[...] = x
        a_ref[...] = jnp.concatenate([A, pad], axis=0)
        b_ref[...] = jnp.concatenate([B, pad], axis=0)

    outs = [jax.ShapeDtypeStruct((N, D), F32),
            jax.ShapeDtypeStruct((NP, D), F32),
            jax.ShapeDtypeStruct((NP, D), F32)]
    return pl.pallas_call(body, out_shape=outs)(
        at_f32, emb, nW, nb, W1a, W1b, b1)


def _update_call(x, s0, s1, deg, sd, lw, residual, nxt, N, NP, D):
    last = nxt is None

    def body(*refs):
        it = iter(refs)
        x_ref, s0r, s1r, degr, sdr = (next(it) for _ in range(5))
        W2r, wr, ber, c2r, U1ar, U1br, ub1r, U2r, ub2r, gr, br = (
            next(it) for _ in range(11))
        if last:
            bidr, pW1r, pb1r, pW2r, pb2r, pW3r, pb3r, pW4r, pb4r = (
                next(it) for _ in range(9))
            out_ref = next(it)
        else:
            nWr, nbr, W1ar, W1br, b1r = (next(it) for _ in range(5))
            xo, ao, bo = (next(it) for _ in range(3))

        x_ = x_ref[...]
        s0W = jnp.dot(s0r[...], W2r[...], preferred_element_type=F32)
        s1W = jnp.dot(s1r[...], W2r[...], preferred_element_type=F32)
        w_ = wr[...]; be = ber[...]; c2 = c2r[...]
        agg = (s1W * w_ + s0W * be +
               sdr[...] * (c2 * w_) + degr[...] * (c2 * be))
        u = jnp.maximum(jnp.dot(agg, U1ar[...], preferred_element_type=F32) +
                        jnp.dot(x_, U1br[...], preferred_element_type=F32) +
                        ub1r[...], 0.0)
        u = jnp.maximum(jnp.dot(u, U2r[...], preferred_element_type=F32) +
                        ub2r[...], 0.0)
        mu = jnp.mean(u, axis=-1, keepdims=True)
        d = u - mu
        var = jnp.mean(d * d, axis=-1, keepdims=True)
        u = d * jax.lax.rsqrt(var + 1e-5) * gr[...] + br[...]
        xn = u if not residual else x_ + u

        if last:
            bid = bidr[...]
            gid = lax.broadcasted_iota(I32, (1, 16), 1).astype(F32)
            ohg = jnp.where(bid == gid, 1.0, 0.0)
            sums = lax.dot_general(ohg, xn, (((0,), (0,)), ((), ())),
                                   preferred_element_type=F32)
            ones = jnp.zeros((N, 1), F32) + 1.0
            counts = lax.dot_general(ohg, ones, (((0,), (0,)), ((), ())),
                                     preferred_element_type=F32)
            pooled = sums / jnp.maximum(counts, 1.0)
            h = jnp.maximum(jnp.dot(pooled, pW1r[...],
                                    preferred_element_type=F32) + pb1r[...], 0.0)
            h = jnp.maximum(jnp.dot(h, pW2r[...],
                                    preferred_element_type=F32) + pb2r[...], 0.0)
            h = jnp.maximum(jnp.dot(h, pW3r[...],
                                    preferred_element_type=F32) + pb3r[...], 0.0)
            out_ref[...] = jnp.dot(h, pW4r[...],
                                   preferred_element_type=F32) + pb4r[...]
        else:
            xt = jnp.dot(xn, nWr[...], preferred_element_type=F32) + nbr[...]
            A = jnp.dot(xt, W1ar[...], preferred_element_type=F32) + b1r[...]
            B = jnp.dot(xt, W1br[...], preferred_element_type=F32)
            pad = jnp.zeros((NP - N, D), F32) - 1e9
            xo[...] = xn
            ao[...] = jnp.concatenate([A, pad], axis=0)
            bo[...] = jnp.concatenate([B, pad], axis=0)

    args = [x, s0, s1, deg, sd,
            lw["W2"], lw["w"], lw["be"], lw["c2"], lw["U1a"], lw["U1b"],
            lw["ub1"], lw["U2"], lw["ub2"], lw["g"], lw["b"]]
    if last:
        args += [lw["bid"], lw["pW1"], lw["pb1"], lw["pW2"], lw["pb2"],
                 lw["pW3"], lw["pb3"], lw["pW4"], lw["pb4"]]
        outs = jax.ShapeDtypeStruct((16, 1), F32)
    else:
        args += [nxt["nW"], nxt["nb"], nxt["W1a"], nxt["W1b"], nxt["b1"]]
        outs = [jax.ShapeDtypeStruct((N, D), F32),
                jax.ShapeDtypeStruct((NP, D), F32),
                jax.ShapeDtypeStruct((NP, D), F32)]
    return pl.pallas_call(body, out_shape=outs)(*args)


def kernel(atom_types, edge_index, batch_ids, distances, emb, node_W, node_b,
           edge_W, edge_b, msg_W1, msg_b1, msg_W2, msg_b2, upd_W1, upd_b1,
           upd_W2, upd_b2, ln_g, ln_b, pW1, pb1, pW2, pb2, pW3, pb3, pW4, pb4):
    N = atom_types.shape[0]
    E = distances.shape[0]
    V, D = emb.shape
    L = node_W.shape[0]
    NP = N + 8

    grain = _NS * _W * _CPW
    E2 = ((E + grain - 1) // grain) * grain
    if (E2 // (_NS * _W * _CPW)) % 2:
        E2 += grain
    pad = E2 - E
    src = jnp.concatenate([edge_index[0], jnp.full((pad,), N, I32)])
    dst = jnp.concatenate([edge_index[1], jnp.full((pad,), N, I32)])
    dist = jnp.concatenate([distances.astype(F32), jnp.zeros((pad,), F32)])
    src2 = src.reshape(E2 // _W, _W)
    dst2 = dst.reshape(E2 // _W, _W)
    dist2 = dist.reshape(E2 // _W, _W)

    at_f32 = atom_types.astype(F32).reshape(N, 1)
    bid_f32 = batch_ids.astype(F32).reshape(N, 1)

    deg = jax.ops.segment_sum(jnp.ones((E,), F32), edge_index[1],
                              num_segments=N).reshape(N, 1)
    sd = jax.ops.segment_sum(distances.astype(F32), edge_index[1],
                             num_segments=N).reshape(N, 1)

    row = lambda v: v.reshape(1, -1)

    def layer_w(i):
        return dict(
            W2=msg_W2[i], w=row(edge_W[i][0]), be=row(edge_b[i]),
            c2=row(msg_b2[i]), U1a=upd_W1[i][:D], U1b=upd_W1[i][D:],
            ub1=row(upd_b1[i]), U2=upd_W2[i], ub2=row(upd_b2[i]),
            g=row(ln_g[i]), b=row(ln_b[i]))

    def prep_w(i):
        return dict(nW=node_W[i], nb=row(node_b[i]), W1a=msg_W1[i][:D],
                    W1b=msg_W1[i][D:], b1=row(msg_b1[i]))

    p0 = prep_w(0)
    x, a_t, b_t = _prep0_call(at_f32, emb, p0["nW"], p0["nb"],
                              p0["W1a"], p0["W1b"], p0["b1"], N, NP, D, V)

    edge = _edge_pass_kernel(N, NP, E2, D)

    for i in range(L):
        s0, s1 = edge(dst2, src2, dist2, a_t, b_t)
        lw = layer_w(i)
        if i < L - 1:
            x, a_t, b_t = _update_call(x, s0, s1, deg, sd, lw,
                                       residual=(i > 0), nxt=prep_w(i + 1),
                                       N=N, NP=NP, D=D)
        else:
            lw.update(bid=bid_f32, pW1=pW1, pb1=row(pb1), pW2=pW2,
                      pb2=row(pb2), pW3=pW3, pb3=row(pb3), pW4=pW4,
                      pb4=row(pb4))
            out = _update_call(x, s0, s1, deg, sd, lw, residual=True,
                               nxt=None, N=N, NP=NP, D=D)
    return out
